# Initial kernel scaffold; baseline (speedup 1.0000x reference)
import jax
import jax.numpy as jnp
from jax.experimental import pallas as pl


def kernel(x, embedding, new_embedding):
    raise NotImplementedError("write your pallas kernel here")


# speedup vs baseline: 5.8115x; 5.8115x over previous
import functools

import jax
import jax.numpy as jnp
from jax import lax
from jax.experimental import pallas as pl
from jax.experimental.pallas import tpu as pltpu
from jax.experimental.pallas import tpu_sc as plsc

NUM_EMB = 100000
NUM_NEW = 128
FEATURES = 2048

_NC = 2
_NS = 16
_NW = _NC * _NS
_L = 16

_B = 4 * 4096
_BPW = _B // _NW
_CH = 16
_NCHUNK = _BPW // _CH
_NWIN = _BPW // _L


def _body(emb_hbm, new_hbm, x_hbm, out_hbm,
          idx_main, ofl_pos, ofl_idx, rows_a, rows_b, sem_a, sem_b):
  wid = lax.axis_index("s") * _NC + lax.axis_index("c")
  base = wid * _BPW

  pltpu.sync_copy(x_hbm.at[pl.ds(base, _BPW)], idx_main)

  def win(i, cnt):
    v = idx_main[pl.ds(i * _L, _L)]
    ovf = v >= NUM_EMB
    m = jnp.where(ovf, 1, 0)
    idx_main[pl.ds(i * _L, _L)] = jnp.where(ovf, NUM_EMB - 1, v)
    incl = plsc.cumsum(m)
    slots = cnt + incl - m
    pos = base + i * _L + lax.iota(jnp.int32, _L)
    plsc.store_scatter(ofl_pos, [slots >> 4, slots & (_L - 1)], pos, mask=ovf)
    plsc.store_scatter(ofl_idx, [slots >> 4, slots & (_L - 1)], v - NUM_EMB,
                       mask=ovf)
    return cnt + incl[_L - 1]
  cnt = lax.fori_loop(0, _NWIN, win, 0)

  bufs = (rows_a, rows_b)
  sems = (sem_a, sem_b)
  pltpu.async_copy(emb_hbm.at[idx_main.at[pl.ds(0, _CH)]], rows_a, sem_a)

  def chunk(g, carry):
    nxt = jnp.minimum(g + 1, _NCHUNK - 1)
    for b in range(2):
      @pl.when(g % 2 == b)
      def _():
        pltpu.make_async_copy(emb_hbm.at[idx_main.at[pl.ds(g * _CH, _CH)]],
                              bufs[b], sems[b]).wait()
        @pl.when(g + 1 < _NCHUNK)
        def _():
          pltpu.async_copy(emb_hbm.at[idx_main.at[pl.ds(nxt * _CH, _CH)]],
                           bufs[1 - b], sems[1 - b])
        pltpu.sync_copy(bufs[b], out_hbm.at[pl.ds(base + g * _CH, _CH)])
    return carry
  lax.fori_loop(0, _NCHUNK, chunk, 0)

  @pl.when(cnt > 0)
  def _():
    p0 = jnp.full((_L,), ofl_pos[0, :][0], jnp.int32)
    i0 = jnp.full((_L,), ofl_idx[0, :][0], jnp.int32)
    slots = cnt + lax.iota(jnp.int32, _L)
    ones = slots >= 0
    plsc.store_scatter(ofl_pos, [slots >> 4, slots & (_L - 1)], p0, mask=ones)
    plsc.store_scatter(ofl_idx, [slots >> 4, slots & (_L - 1)], i0, mask=ones)

    def fix(n, carry):
      pltpu.async_copy(new_hbm.at[ofl_idx.at[n]], rows_a, sem_a).wait()
      pltpu.async_copy(rows_a, out_hbm.at[ofl_pos.at[n]], sem_a).wait()
      return carry
    lax.fori_loop(0, (cnt + _L - 1) // _L, fix, 0)


@jax.jit
def kernel(x, embedding, new_embedding):
  mesh = plsc.VectorSubcoreMesh(core_axis_name="c", subcore_axis_name="s")
  call = functools.partial(
      pl.kernel,
      out_type=jax.ShapeDtypeStruct((_B, FEATURES), jnp.float32),
      mesh=mesh,
      compiler_params=pltpu.CompilerParams(needs_layout_passes=False),
      scratch_types=[
          pltpu.VMEM((_BPW,), jnp.int32),
          pltpu.VMEM((_NWIN + 2, _L), jnp.int32),
          pltpu.VMEM((_NWIN + 2, _L), jnp.int32),
          pltpu.VMEM((_CH, FEATURES), jnp.float32),
          pltpu.VMEM((_CH, FEATURES), jnp.float32),
          pltpu.SemaphoreType.DMA,
          pltpu.SemaphoreType.DMA,
      ],
  )(_body)
  out = call(embedding, new_embedding, x.reshape(_B))
  return out.reshape(x.shape + (FEATURES,))


# speedup vs baseline: 5.9376x; 1.0217x over previous
import functools

import jax
import jax.numpy as jnp
from jax import lax
from jax.experimental import pallas as pl
from jax.experimental.pallas import tpu as pltpu
from jax.experimental.pallas import tpu_sc as plsc

NUM_EMB = 100000
NUM_NEW = 128
FEATURES = 2048

_NC = 2
_NS = 16
_NW = _NC * _NS
_L = 16

_B = 4 * 4096
_BPW = _B // _NW
_CH = 16
_NBUF = 3
_NCHUNK = _BPW // _CH
_NWIN = _BPW // _L


def _body(emb_hbm, new_hbm, x_hbm, out_hbm,
          idx_main, ofl_pos, ofl_idx, rows_a, rows_b, rows_c,
          gsem_a, gsem_b, gsem_c, wsem_a, wsem_b, wsem_c):
  wid = lax.axis_index("s") * _NC + lax.axis_index("c")
  base = wid * _BPW

  pltpu.sync_copy(x_hbm.at[pl.ds(base, _BPW)], idx_main)

  def win(i, cnt):
    v = idx_main[pl.ds(i * _L, _L)]
    ovf = v >= NUM_EMB
    m = jnp.where(ovf, 1, 0)
    idx_main[pl.ds(i * _L, _L)] = jnp.where(ovf, NUM_EMB - 1, v)
    incl = plsc.cumsum(m)
    slots = cnt + incl - m
    pos = base + i * _L + lax.iota(jnp.int32, _L)
    plsc.store_scatter(ofl_pos, [slots >> 4, slots & (_L - 1)], pos, mask=ovf)
    plsc.store_scatter(ofl_idx, [slots >> 4, slots & (_L - 1)], v - NUM_EMB,
                       mask=ovf)
    return cnt + incl[_L - 1]
  cnt = lax.fori_loop(0, _NWIN, win, 0)

  bufs = (rows_a, rows_b, rows_c)
  gsems = (gsem_a, gsem_b, gsem_c)
  wsems = (wsem_a, wsem_b, wsem_c)
  for g in range(_NBUF):
    pltpu.async_copy(emb_hbm.at[idx_main.at[pl.ds(g * _CH, _CH)]],
                     bufs[g], gsems[g])

  def step(gg, carry):
    for b in range(_NBUF):
      g = gg * _NBUF + b
      pltpu.make_async_copy(emb_hbm.at[idx_main.at[pl.ds(g * _CH, _CH)]],
                            bufs[b], gsems[b]).wait()
      pltpu.async_copy(bufs[b], out_hbm.at[pl.ds(base + g * _CH, _CH)],
                       wsems[b])
      pb = (b - 1) % _NBUF
      @pl.when((g >= 1) & (g - 1 + _NBUF < _NCHUNK))
      def _():
        pg = g - 1
        pltpu.make_async_copy(bufs[pb],
                              out_hbm.at[pl.ds(base + pg * _CH, _CH)],
                              wsems[pb]).wait()
        pltpu.async_copy(
            emb_hbm.at[idx_main.at[pl.ds((pg + _NBUF) * _CH, _CH)]],
            bufs[pb], gsems[pb])
    return carry
  lax.fori_loop(0, _NCHUNK // _NBUF, step, 0)
  for g in range((_NCHUNK // _NBUF) * _NBUF, _NCHUNK):
    b = g % _NBUF
    pltpu.make_async_copy(emb_hbm.at[idx_main.at[pl.ds(g * _CH, _CH)]],
                          bufs[b], gsems[b]).wait()
    pltpu.async_copy(bufs[b], out_hbm.at[pl.ds(base + g * _CH, _CH)],
                     wsems[b])
  for g in range(_NCHUNK - _NBUF, _NCHUNK):
    b = g % _NBUF
    pltpu.make_async_copy(bufs[b],
                          out_hbm.at[pl.ds(base + g * _CH, _CH)],
                          wsems[b]).wait()

  @pl.when(cnt > 0)
  def _():
    p0 = jnp.full((_L,), ofl_pos[0, :][0], jnp.int32)
    i0 = jnp.full((_L,), ofl_idx[0, :][0], jnp.int32)
    slots = cnt + lax.iota(jnp.int32, _L)
    ones = slots >= 0
    plsc.store_scatter(ofl_pos, [slots >> 4, slots & (_L - 1)], p0, mask=ones)
    plsc.store_scatter(ofl_idx, [slots >> 4, slots & (_L - 1)], i0, mask=ones)

    def fix(n, carry):
      pltpu.async_copy(new_hbm.at[ofl_idx.at[n]], rows_a, gsem_a).wait()
      pltpu.async_copy(rows_a, out_hbm.at[ofl_pos.at[n]], gsem_a).wait()
      return carry
    lax.fori_loop(0, (cnt + _L - 1) // _L, fix, 0)


@jax.jit
def kernel(x, embedding, new_embedding):
  mesh = plsc.VectorSubcoreMesh(core_axis_name="c", subcore_axis_name="s")
  call = functools.partial(
      pl.kernel,
      out_type=jax.ShapeDtypeStruct((_B, FEATURES), jnp.float32),
      mesh=mesh,
      compiler_params=pltpu.CompilerParams(needs_layout_passes=False),
      scratch_types=[
          pltpu.VMEM((_BPW,), jnp.int32),
          pltpu.VMEM((_NWIN + 2, _L), jnp.int32),
          pltpu.VMEM((_NWIN + 2, _L), jnp.int32),
          pltpu.VMEM((_CH, FEATURES), jnp.float32),
          pltpu.VMEM((_CH, FEATURES), jnp.float32),
          pltpu.VMEM((_CH, FEATURES), jnp.float32),
          pltpu.SemaphoreType.DMA,
          pltpu.SemaphoreType.DMA,
          pltpu.SemaphoreType.DMA,
          pltpu.SemaphoreType.DMA,
          pltpu.SemaphoreType.DMA,
          pltpu.SemaphoreType.DMA,
      ],
  )(_body)
  out = call(embedding, new_embedding, x.reshape(_B))
  return out.reshape(x.shape + (FEATURES,))


# speedup vs baseline: 6.0054x; 1.0114x over previous
import functools

import jax
import jax.numpy as jnp
from jax import lax
from jax.experimental import pallas as pl
from jax.experimental.pallas import tpu as pltpu
from jax.experimental.pallas import tpu_sc as plsc

NUM_EMB = 100000
NUM_NEW = 128
FEATURES = 2048

_NC = 2
_NS = 16
_NW = _NC * _NS
_L = 16

_B = 4 * 4096
_BPW = _B // _NW
_CH = 8
_NBUF = 4
_NCHUNK = _BPW // _CH
_NWIN = _BPW // _L


def _body(emb_hbm, new_hbm, x_hbm, out_hbm,
          idx_main, ofl_pos, ofl_idx, rows_a, rows_b, rows_c, rows_d,
          rows_fix,
          gsem_a, gsem_b, gsem_c, gsem_d, wsem_a, wsem_b, wsem_c, wsem_d):
  wid = lax.axis_index("s") * _NC + lax.axis_index("c")
  base = wid * _BPW

  pltpu.sync_copy(x_hbm.at[pl.ds(base, _BPW)], idx_main)

  def win(i, cnt):
    v = idx_main[pl.ds(i * _L, _L)]
    ovf = v >= NUM_EMB
    m = jnp.where(ovf, 1, 0)
    idx_main[pl.ds(i * _L, _L)] = jnp.where(ovf, NUM_EMB - 1, v)
    incl = plsc.cumsum(m)
    slots = cnt + incl - m
    pos = base + i * _L + lax.iota(jnp.int32, _L)
    plsc.store_scatter(ofl_pos, [slots >> 4, slots & (_L - 1)], pos, mask=ovf)
    plsc.store_scatter(ofl_idx, [slots >> 4, slots & (_L - 1)], v - NUM_EMB,
                       mask=ovf)
    return cnt + incl[_L - 1]
  cnt = lax.fori_loop(0, _NWIN, win, 0)

  bufs = (rows_a, rows_b, rows_c, rows_d)
  gsems = (gsem_a, gsem_b, gsem_c, gsem_d)
  wsems = (wsem_a, wsem_b, wsem_c, wsem_d)
  for g in range(_NBUF):
    pltpu.async_copy(emb_hbm.at[idx_main.at[pl.ds(g * _CH, _CH)]],
                     bufs[g], gsems[g])

  def step(gg, carry):
    for b in range(_NBUF):
      g = gg * _NBUF + b
      pltpu.make_async_copy(emb_hbm.at[idx_main.at[pl.ds(g * _CH, _CH)]],
                            bufs[b], gsems[b]).wait()
      pltpu.async_copy(bufs[b], out_hbm.at[pl.ds(base + g * _CH, _CH)],
                       wsems[b])
      pb = (b - 1) % _NBUF
      @pl.when((g >= 1) & (g - 1 + _NBUF < _NCHUNK))
      def _():
        pg = g - 1
        pltpu.make_async_copy(bufs[pb],
                              out_hbm.at[pl.ds(base + pg * _CH, _CH)],
                              wsems[pb]).wait()
        pltpu.async_copy(
            emb_hbm.at[idx_main.at[pl.ds((pg + _NBUF) * _CH, _CH)]],
            bufs[pb], gsems[pb])
    return carry
  lax.fori_loop(0, _NCHUNK // _NBUF, step, 0)
  for g in range((_NCHUNK // _NBUF) * _NBUF, _NCHUNK):
    b = g % _NBUF
    pltpu.make_async_copy(emb_hbm.at[idx_main.at[pl.ds(g * _CH, _CH)]],
                          bufs[b], gsems[b]).wait()
    pltpu.async_copy(bufs[b], out_hbm.at[pl.ds(base + g * _CH, _CH)],
                     wsems[b])
  for g in range(_NCHUNK - _NBUF, _NCHUNK):
    b = g % _NBUF
    pltpu.make_async_copy(bufs[b],
                          out_hbm.at[pl.ds(base + g * _CH, _CH)],
                          wsems[b]).wait()

  @pl.when(cnt > 0)
  def _():
    p0 = jnp.full((_L,), ofl_pos[0, :][0], jnp.int32)
    i0 = jnp.full((_L,), ofl_idx[0, :][0], jnp.int32)
    slots = cnt + lax.iota(jnp.int32, _L)
    ones = slots >= 0
    plsc.store_scatter(ofl_pos, [slots >> 4, slots & (_L - 1)], p0, mask=ones)
    plsc.store_scatter(ofl_idx, [slots >> 4, slots & (_L - 1)], i0, mask=ones)

    def fix(n, carry):
      pltpu.async_copy(new_hbm.at[ofl_idx.at[n]], rows_fix, gsem_a).wait()
      pltpu.async_copy(rows_fix, out_hbm.at[ofl_pos.at[n]], gsem_a).wait()
      return carry
    lax.fori_loop(0, (cnt + _L - 1) // _L, fix, 0)


@jax.jit
def kernel(x, embedding, new_embedding):
  mesh = plsc.VectorSubcoreMesh(core_axis_name="c", subcore_axis_name="s")
  call = functools.partial(
      pl.kernel,
      out_type=jax.ShapeDtypeStruct((_B, FEATURES), jnp.float32),
      mesh=mesh,
      compiler_params=pltpu.CompilerParams(needs_layout_passes=False),
      scratch_types=[
          pltpu.VMEM((_BPW,), jnp.int32),
          pltpu.VMEM((_NWIN + 2, _L), jnp.int32),
          pltpu.VMEM((_NWIN + 2, _L), jnp.int32),
          pltpu.VMEM((_CH, FEATURES), jnp.float32),
          pltpu.VMEM((_CH, FEATURES), jnp.float32),
          pltpu.VMEM((_CH, FEATURES), jnp.float32),
          pltpu.VMEM((_CH, FEATURES), jnp.float32),
          pltpu.VMEM((_L, FEATURES), jnp.float32),
          pltpu.SemaphoreType.DMA,
          pltpu.SemaphoreType.DMA,
          pltpu.SemaphoreType.DMA,
          pltpu.SemaphoreType.DMA,
          pltpu.SemaphoreType.DMA,
          pltpu.SemaphoreType.DMA,
          pltpu.SemaphoreType.DMA,
          pltpu.SemaphoreType.DMA,
      ],
  )(_body)
  out = call(embedding, new_embedding, x.reshape(_B))
  return out.reshape(x.shape + (FEATURES,))


# speedup vs baseline: 6.1361x; 1.0218x over previous
import functools

import jax
import jax.numpy as jnp
from jax import lax
from jax.experimental import pallas as pl
from jax.experimental.pallas import tpu as pltpu
from jax.experimental.pallas import tpu_sc as plsc

NUM_EMB = 100000
NUM_NEW = 128
FEATURES = 2048

_NC = 2
_NS = 16
_NW = _NC * _NS
_L = 16

_B = 4 * 4096
_BPW = _B // _NW
_CH = 8
_NBUF = 4
_NCHUNK = _BPW // _CH
_NWIN = _BPW // _L


def _body(emb_hbm, new_hbm, x_hbm, out_hbm,
          idx_main, ofl_pos, ofl_idx, rows_a, rows_b, rows_c, rows_d,
          rows_fix,
          gsem_a, gsem_b, gsem_c, gsem_d, wsem_a, wsem_b, wsem_c, wsem_d,
          fsem):
  wid = lax.axis_index("s") * _NC + lax.axis_index("c")
  base = wid * _BPW

  pltpu.sync_copy(x_hbm.at[pl.ds(base, _BPW)], idx_main)

  def win(i, cnt):
    v = idx_main[pl.ds(i * _L, _L)]
    ovf = v >= NUM_EMB
    m = jnp.where(ovf, 1, 0)
    idx_main[pl.ds(i * _L, _L)] = jnp.where(ovf, NUM_EMB - 1, v)
    incl = plsc.cumsum(m)
    slots = cnt + incl - m
    pos = base + i * _L + lax.iota(jnp.int32, _L)
    plsc.store_scatter(ofl_pos, [slots >> 4, slots & (_L - 1)], pos, mask=ovf)
    plsc.store_scatter(ofl_idx, [slots >> 4, slots & (_L - 1)], v - NUM_EMB,
                       mask=ovf)
    return cnt + incl[_L - 1]

  _PWIN = (_NBUF * _CH) // _L
  cnt0 = lax.fori_loop(0, _PWIN, win, 0)

  bufs = (rows_a, rows_b, rows_c, rows_d)
  gsems = (gsem_a, gsem_b, gsem_c, gsem_d)
  wsems = (wsem_a, wsem_b, wsem_c, wsem_d)
  for g in range(_NBUF):
    pltpu.async_copy(emb_hbm.at[idx_main.at[pl.ds(g * _CH, _CH)]],
                     bufs[g], gsems[g])

  cnt = lax.fori_loop(_PWIN, _NWIN, win, cnt0)

  @pl.when(cnt > 0)
  def _():
    p0 = jnp.full((_L,), ofl_pos[0, :][0], jnp.int32)
    i0 = jnp.full((_L,), ofl_idx[0, :][0], jnp.int32)
    slots = cnt + lax.iota(jnp.int32, _L)
    ones = slots >= 0
    plsc.store_scatter(ofl_pos, [slots >> 4, slots & (_L - 1)], p0, mask=ones)
    plsc.store_scatter(ofl_idx, [slots >> 4, slots & (_L - 1)], i0, mask=ones)
    pltpu.async_copy(new_hbm.at[ofl_idx.at[0]], rows_fix, fsem)

  def step(gg, carry):
    for b in range(_NBUF):
      g = gg * _NBUF + b
      pltpu.make_async_copy(emb_hbm.at[idx_main.at[pl.ds(g * _CH, _CH)]],
                            bufs[b], gsems[b]).wait()
      pltpu.async_copy(bufs[b], out_hbm.at[pl.ds(base + g * _CH, _CH)],
                       wsems[b])
      pb = (b - 1) % _NBUF
      @pl.when((g >= 1) & (g - 1 + _NBUF < _NCHUNK))
      def _():
        pg = g - 1
        pltpu.make_async_copy(bufs[pb],
                              out_hbm.at[pl.ds(base + pg * _CH, _CH)],
                              wsems[pb]).wait()
        pltpu.async_copy(
            emb_hbm.at[idx_main.at[pl.ds((pg + _NBUF) * _CH, _CH)]],
            bufs[pb], gsems[pb])
    return carry
  lax.fori_loop(0, _NCHUNK // _NBUF, step, 0)
  for g in range((_NCHUNK // _NBUF) * _NBUF, _NCHUNK):
    b = g % _NBUF
    pltpu.make_async_copy(emb_hbm.at[idx_main.at[pl.ds(g * _CH, _CH)]],
                          bufs[b], gsems[b]).wait()
    pltpu.async_copy(bufs[b], out_hbm.at[pl.ds(base + g * _CH, _CH)],
                     wsems[b])
  for g in range(_NCHUNK - _NBUF, _NCHUNK):
    b = g % _NBUF
    pltpu.make_async_copy(bufs[b],
                          out_hbm.at[pl.ds(base + g * _CH, _CH)],
                          wsems[b]).wait()

  @pl.when(cnt > 0)
  def _():
    nfix = (cnt + _L - 1) // _L

    def fix(n, carry):
      pltpu.make_async_copy(new_hbm.at[ofl_idx.at[n]], rows_fix, fsem).wait()
      pltpu.async_copy(rows_fix, out_hbm.at[ofl_pos.at[n]], gsem_a).wait()
      @pl.when(n + 1 < nfix)
      def _():
        pltpu.async_copy(new_hbm.at[ofl_idx.at[n + 1]], rows_fix, fsem)
      return carry
    lax.fori_loop(0, nfix, fix, 0)


@jax.jit
def kernel(x, embedding, new_embedding):
  mesh = plsc.VectorSubcoreMesh(core_axis_name="c", subcore_axis_name="s")
  call = functools.partial(
      pl.kernel,
      out_type=jax.ShapeDtypeStruct((_B, FEATURES), jnp.float32),
      mesh=mesh,
      compiler_params=pltpu.CompilerParams(
          needs_layout_passes=False,
          disable_bounds_checks=True,
          disable_semaphore_checks=True,
      ),
      scratch_types=[
          pltpu.VMEM((_BPW,), jnp.int32),
          pltpu.VMEM((_NWIN + 2, _L), jnp.int32),
          pltpu.VMEM((_NWIN + 2, _L), jnp.int32),
          pltpu.VMEM((_CH, FEATURES), jnp.float32),
          pltpu.VMEM((_CH, FEATURES), jnp.float32),
          pltpu.VMEM((_CH, FEATURES), jnp.float32),
          pltpu.VMEM((_CH, FEATURES), jnp.float32),
          pltpu.VMEM((_L, FEATURES), jnp.float32),
          pltpu.SemaphoreType.DMA,
          pltpu.SemaphoreType.DMA,
          pltpu.SemaphoreType.DMA,
          pltpu.SemaphoreType.DMA,
          pltpu.SemaphoreType.DMA,
          pltpu.SemaphoreType.DMA,
          pltpu.SemaphoreType.DMA,
          pltpu.SemaphoreType.DMA,
          pltpu.SemaphoreType.DMA,
      ],
  )(_body)
  out = call(embedding, new_embedding, x.reshape(_B))
  return out.reshape(x.shape + (FEATURES,))


# speedup vs baseline: 6.1647x; 1.0047x over previous
import functools

import jax
import jax.numpy as jnp
from jax import lax
from jax.experimental import pallas as pl
from jax.experimental.pallas import tpu as pltpu
from jax.experimental.pallas import tpu_sc as plsc

NUM_EMB = 100000
NUM_NEW = 128
FEATURES = 2048

_NC = 2
_NS = 16
_NW = _NC * _NS
_L = 16

_B = 4 * 4096
_BPW = _B // _NW
_CH = 8
_NBUF = 4
_NCHUNK = _BPW // _CH
_NWIN = _BPW // _L


def _body(emb_hbm, new_hbm, x_hbm, out_hbm,
          idx_main, ofl_pos, ofl_idx, rows_a, rows_b, rows_c, rows_d,
          rows_fix,
          gsem_a, gsem_b, gsem_c, gsem_d, wsem_a, wsem_b, wsem_c, wsem_d,
          fsem):
  wid = lax.axis_index("s") * _NC + lax.axis_index("c")
  base = wid * _BPW

  pltpu.sync_copy(x_hbm.at[pl.ds(base, _BPW)], idx_main)

  def win(i, cnt):
    v = idx_main[pl.ds(i * _L, _L)]
    ovf = v >= NUM_EMB
    m = jnp.where(ovf, 1, 0)
    idx_main[pl.ds(i * _L, _L)] = jnp.where(ovf, NUM_EMB - 1, v)
    incl = plsc.cumsum(m)
    slots = cnt + incl - m
    pos = base + i * _L + lax.iota(jnp.int32, _L)
    plsc.store_scatter(ofl_pos, [slots >> 4, slots & (_L - 1)], pos, mask=ovf)
    plsc.store_scatter(ofl_idx, [slots >> 4, slots & (_L - 1)], v - NUM_EMB,
                       mask=ovf)
    return cnt + incl[_L - 1]

  _PWIN = (_NBUF * _CH) // _L
  cnt0 = lax.fori_loop(0, _PWIN, win, 0)

  bufs = (rows_a, rows_b, rows_c, rows_d)
  gsems = (gsem_a, gsem_b, gsem_c, gsem_d)
  wsems = (wsem_a, wsem_b, wsem_c, wsem_d)
  for g in range(_NBUF):
    pltpu.async_copy(emb_hbm.at[idx_main.at[pl.ds(g * _CH, _CH)]],
                     bufs[g], gsems[g])

  cnt = lax.fori_loop(_PWIN, _NWIN, win, cnt0)

  @pl.when(cnt > 0)
  def _():
    p0 = jnp.full((_L,), ofl_pos[0, :][0], jnp.int32)
    i0 = jnp.full((_L,), ofl_idx[0, :][0], jnp.int32)
    slots = cnt + lax.iota(jnp.int32, _L)
    ones = slots >= 0
    plsc.store_scatter(ofl_pos, [slots >> 4, slots & (_L - 1)], p0, mask=ones)
    plsc.store_scatter(ofl_idx, [slots >> 4, slots & (_L - 1)], i0, mask=ones)
    pltpu.async_copy(new_hbm.at[ofl_idx.at[0]], rows_fix, fsem)

  def step(gg, carry):
    for b in range(_NBUF):
      g = gg * _NBUF + b
      pltpu.make_async_copy(emb_hbm.at[idx_main.at[pl.ds(g * _CH, _CH)]],
                            bufs[b], gsems[b]).wait()
      pltpu.async_copy(bufs[b], out_hbm.at[pl.ds(base + g * _CH, _CH)],
                       wsems[b])
      pb = (b - 1) % _NBUF
      @pl.when((g >= 1) & (g - 1 + _NBUF < _NCHUNK))
      def _():
        pg = g - 1
        pltpu.make_async_copy(bufs[pb],
                              out_hbm.at[pl.ds(base + pg * _CH, _CH)],
                              wsems[pb]).wait()
        pltpu.async_copy(
            emb_hbm.at[idx_main.at[pl.ds((pg + _NBUF) * _CH, _CH)]],
            bufs[pb], gsems[pb])
    return carry
  lax.fori_loop(0, _NCHUNK // _NBUF, step, 0)
  for g in range((_NCHUNK // _NBUF) * _NBUF, _NCHUNK):
    b = g % _NBUF
    pltpu.make_async_copy(emb_hbm.at[idx_main.at[pl.ds(g * _CH, _CH)]],
                          bufs[b], gsems[b]).wait()
    pltpu.async_copy(bufs[b], out_hbm.at[pl.ds(base + g * _CH, _CH)],
                     wsems[b])
  for g in range(_NCHUNK - _NBUF, _NCHUNK):
    b = g % _NBUF
    pltpu.make_async_copy(bufs[b],
                          out_hbm.at[pl.ds(base + g * _CH, _CH)],
                          wsems[b]).wait()

  @pl.when(cnt > 0)
  def _():
    nfix = (cnt + _L - 1) // _L

    def fix(n, carry):
      pltpu.make_async_copy(new_hbm.at[ofl_idx.at[n]], rows_fix, fsem).wait()
      pltpu.async_copy(rows_fix, out_hbm.at[ofl_pos.at[n]], gsem_a).wait()
      @pl.when(n + 1 < nfix)
      def _():
        pltpu.async_copy(new_hbm.at[ofl_idx.at[n + 1]], rows_fix, fsem)
      return carry
    lax.fori_loop(0, nfix, fix, 0)


@jax.jit
def kernel(x, embedding, new_embedding):
  mesh = plsc.VectorSubcoreMesh(core_axis_name="c", subcore_axis_name="s")
  call = functools.partial(
      pl.kernel,
      out_type=jax.ShapeDtypeStruct((_B, FEATURES), jnp.float32),
      mesh=mesh,
      compiler_params=pltpu.CompilerParams(
          needs_layout_passes=False,
          disable_bounds_checks=True,
          disable_semaphore_checks=True,
          skip_device_barrier=True,
      ),
      scratch_types=[
          pltpu.VMEM((_BPW,), jnp.int32),
          pltpu.VMEM((_NWIN + 2, _L), jnp.int32),
          pltpu.VMEM((_NWIN + 2, _L), jnp.int32),
          pltpu.VMEM((_CH, FEATURES), jnp.float32),
          pltpu.VMEM((_CH, FEATURES), jnp.float32),
          pltpu.VMEM((_CH, FEATURES), jnp.float32),
          pltpu.VMEM((_CH, FEATURES), jnp.float32),
          pltpu.VMEM((_L, FEATURES), jnp.float32),
          pltpu.SemaphoreType.DMA,
          pltpu.SemaphoreType.DMA,
          pltpu.SemaphoreType.DMA,
          pltpu.SemaphoreType.DMA,
          pltpu.SemaphoreType.DMA,
          pltpu.SemaphoreType.DMA,
          pltpu.SemaphoreType.DMA,
          pltpu.SemaphoreType.DMA,
          pltpu.SemaphoreType.DMA,
      ],
  )(_body)
  out = call(embedding, new_embedding, x.reshape(_B))
  return out.reshape(x.shape + (FEATURES,))
